# baseline (speedup 1.0000x reference)
import functools

import jax
import jax.numpy as jnp
from jax import lax
from jax.experimental import pallas as pl
from jax.experimental.pallas import tpu as pltpu
from jax.experimental.pallas import tpu_sc as plsc

_NUM_CORES = 2
_NUM_SUBCORES = 16
_N_WORKERS = _NUM_CORES * _NUM_SUBCORES

_CHUNK = 800
_LANES = 16

_ROW_BLOCK = 4000


@functools.lru_cache(maxsize=None)
def _make_gather(batch, n_rows, d):
  n_chunks = batch // _CHUNK
  assert batch % _CHUNK == 0
  max_rounds = -(-n_chunks // _N_WORKERS)
  assert max_rounds >= 4 and n_chunks >= (max_rounds - 2) * _N_WORKERS
  mesh = plsc.VectorSubcoreMesh(core_axis_name="c", subcore_axis_name="s")

  @functools.partial(
      pl.kernel,
      mesh=mesh,
      compiler_params=pltpu.CompilerParams(use_tc_tiling_on_sc=False),
      out_type=jax.ShapeDtypeStruct((batch, d), jnp.float32),
      scratch_types=[
          pltpu.VMEM((_CHUNK,), jnp.int32),
          pltpu.VMEM((_CHUNK,), jnp.int32),
          pltpu.VMEM((_CHUNK, d), jnp.float32),
          pltpu.VMEM((_CHUNK, d), jnp.float32),
          pltpu.VMEM((_LANES,), jnp.int32),
          pltpu.SemaphoreType.DMA,
          pltpu.SemaphoreType.DMA,
          pltpu.SemaphoreType.DMA,
          pltpu.SemaphoreType.DMA,
      ],
  )
  def gather(ids_hbm, gate_hbm, table_hbm, out_hbm,
             idx_v0, idx_v1, rows_v0, rows_v1, gate_v,
             gsem0, gsem1, wsem0, wsem1):
    idx_v = (idx_v0, idx_v1)
    rows_v = (rows_v0, rows_v1)
    gsem = (gsem0, gsem1)
    wsem = (wsem0, wsem1)

    w = lax.axis_index("s") * _NUM_CORES + lax.axis_index("c")
    pltpu.sync_copy(gate_hbm, gate_v)
    use_ids = gate_v[...] > 0
    sentinel = jnp.full((_LANES,), n_rows - 1, jnp.int32)

    def chunk_off(j):
      return (w + _N_WORKERS * j) * _CHUNK

    def has_chunk(j):
      return w + _N_WORKERS * j < n_chunks

    def load_select(j, b):
      pltpu.sync_copy(ids_hbm.at[pl.ds(chunk_off(j), _CHUNK)], idx_v[b])
      for i in range(_CHUNK // _LANES):
        sl = pl.ds(i * _LANES, _LANES)
        idx_v[b][sl] = jnp.where(use_ids, idx_v[b][sl], sentinel)

    def g_start(b):
      pltpu.async_copy(table_hbm.at[idx_v[b]], rows_v[b], gsem[b])

    def g_wait(b):
      pltpu.make_async_copy(table_hbm.at[idx_v[b]], rows_v[b],
                            gsem[b]).wait()

    def w_dst(j):
      return out_hbm.at[pl.ds(chunk_off(j), _CHUNK)]

    def w_start(j, b):
      pltpu.async_copy(rows_v[b], w_dst(j), wsem[b])

    def w_wait(j, b):
      pltpu.make_async_copy(rows_v[b], w_dst(j), wsem[b]).wait()

    load_select(0, 0)
    g_start(0)
    load_select(1, 1)
    g_start(1)
    g_wait(0)
    w_start(0, 0)
    g_wait(1)
    w_start(1, 1)
    for j in range(2, max_rounds):
      b = j % 2
      cond = has_chunk(j)

      @pl.when(cond)
      def _prep():
        load_select(j, b)

      w_wait(j - 2, b)

      @pl.when(cond)
      def _fire():
        g_start(b)
        g_wait(b)
        w_start(j, b)

    for j in range(max_rounds - 2, max_rounds):
      b = j % 2

      @pl.when(has_chunk(j))
      def _drain():
        w_wait(j, b)

  return gather


def _feats_body(feats_ref, out_ref):
  out_ref[...] = feats_ref[...]


def _emb_body(prev_ref, rows_ref, w_ref, b_ref, out_ref):
  del prev_ref
  emb = lax.dot_general(
      rows_ref[...], w_ref[...],
      (((1,), (1,)), ((), ())),
      preferred_element_type=jnp.float32,
  ) + b_ref[...]
  out_ref[...] = jnp.concatenate([emb, jnp.zeros_like(emb)], axis=1)


def kernel(ids, feats, layer_idx, table, W, b):
  batch, in_dim = feats.shape
  d = table.shape[1]
  out_shape = jax.ShapeDtypeStruct((batch, in_dim + d), jnp.float32)

  gate = jnp.broadcast_to(
      jnp.asarray(layer_idx, jnp.int32).reshape(()), (_LANES,))
  rows = _make_gather(batch, table.shape[0], d)(
      ids.astype(jnp.int32), gate, table)

  grid = batch // _ROW_BLOCK
  out0 = pl.pallas_call(
      _feats_body,
      grid=(grid,),
      in_specs=[pl.BlockSpec((_ROW_BLOCK, in_dim), lambda i: (i, 0))],
      out_specs=pl.BlockSpec((_ROW_BLOCK, in_dim), lambda i: (i, 0)),
      out_shape=out_shape,
  )(feats)
  out = pl.pallas_call(
      _emb_body,
      grid=(grid,),
      in_specs=[
          pl.BlockSpec((8, 128), lambda i: (0, 0)),
          pl.BlockSpec((_ROW_BLOCK, d), lambda i: (i, 0)),
          pl.BlockSpec((d, d), lambda i: (0, 0)),
          pl.BlockSpec((1, d), lambda i: (0, 0)),
      ],
      out_specs=pl.BlockSpec((_ROW_BLOCK, in_dim), lambda i: (i, 1)),
      out_shape=out_shape,
      input_output_aliases={0: 0},
  )(out0, rows, W, b.reshape(1, d))
  return out


# speedup vs baseline: 1.0123x; 1.0123x over previous
import functools

import jax
import jax.numpy as jnp
from jax import lax
from jax.experimental import pallas as pl
from jax.experimental.pallas import tpu as pltpu
from jax.experimental.pallas import tpu_sc as plsc

_NUM_CORES = 2
_NUM_SUBCORES = 16
_N_WORKERS = _NUM_CORES * _NUM_SUBCORES

_CHUNK = 800
_LANES = 16

_ROW_BLOCK = 4000


@functools.lru_cache(maxsize=None)
def _make_gather(batch, n_rows, d):
  n_chunks = batch // _CHUNK
  assert batch % _CHUNK == 0
  max_rounds = -(-n_chunks // _N_WORKERS)
  assert max_rounds >= 4 and n_chunks >= (max_rounds - 2) * _N_WORKERS
  mesh = plsc.VectorSubcoreMesh(core_axis_name="c", subcore_axis_name="s")

  @functools.partial(
      pl.kernel,
      mesh=mesh,
      compiler_params=pltpu.CompilerParams(use_tc_tiling_on_sc=False),
      out_type=jax.ShapeDtypeStruct((batch, d), jnp.float32),
      scratch_types=[
          pltpu.VMEM((_CHUNK,), jnp.int32),
          pltpu.VMEM((_CHUNK,), jnp.int32),
          pltpu.VMEM((_CHUNK, d), jnp.float32),
          pltpu.VMEM((_CHUNK, d), jnp.float32),
          pltpu.VMEM((_LANES,), jnp.int32),
          pltpu.SemaphoreType.DMA,
          pltpu.SemaphoreType.DMA,
          pltpu.SemaphoreType.DMA,
          pltpu.SemaphoreType.DMA,
      ],
  )
  def gather(ids_hbm, gate_hbm, table_hbm, out_hbm,
             idx_v0, idx_v1, rows_v0, rows_v1, gate_v,
             gsem0, gsem1, wsem0, wsem1):
    idx_v = (idx_v0, idx_v1)
    rows_v = (rows_v0, rows_v1)
    gsem = (gsem0, gsem1)
    wsem = (wsem0, wsem1)

    w = lax.axis_index("s") * _NUM_CORES + lax.axis_index("c")
    pltpu.sync_copy(gate_hbm, gate_v)
    use_ids = gate_v[...] > 0
    sentinel = jnp.full((_LANES,), n_rows - 1, jnp.int32)

    def chunk_off(j):
      return (w + _N_WORKERS * j) * _CHUNK

    def has_chunk(j):
      return w + _N_WORKERS * j < n_chunks

    def load_select(j, b):
      pltpu.sync_copy(ids_hbm.at[pl.ds(chunk_off(j), _CHUNK)], idx_v[b])
      for i in range(_CHUNK // _LANES):
        sl = pl.ds(i * _LANES, _LANES)
        idx_v[b][sl] = jnp.where(use_ids, idx_v[b][sl], sentinel)

    def g_start(b):
      pltpu.async_copy(table_hbm.at[idx_v[b]], rows_v[b], gsem[b])

    def g_wait(b):
      pltpu.make_async_copy(table_hbm.at[idx_v[b]], rows_v[b],
                            gsem[b]).wait()

    def w_dst(j):
      return out_hbm.at[pl.ds(chunk_off(j), _CHUNK)]

    def w_start(j, b):
      pltpu.async_copy(rows_v[b], w_dst(j), wsem[b])

    def w_wait(j, b):
      pltpu.make_async_copy(rows_v[b], w_dst(j), wsem[b]).wait()

    load_select(0, 0)
    g_start(0)
    load_select(1, 1)
    g_start(1)
    g_wait(0)
    w_start(0, 0)
    g_wait(1)
    w_start(1, 1)
    for j in range(2, max_rounds):
      b = j % 2
      cond = has_chunk(j)

      @pl.when(cond)
      def _prep():
        load_select(j, b)

      w_wait(j - 2, b)

      @pl.when(cond)
      def _fire():
        g_start(b)
        g_wait(b)
        w_start(j, b)

    for j in range(max_rounds - 2, max_rounds):
      b = j % 2

      @pl.when(has_chunk(j))
      def _drain():
        w_wait(j, b)

  return gather


def _fused_body(feats_ref, rows_ref, w_ref, b_ref, out_ref):
  in_dim = feats_ref.shape[1]
  d = w_ref.shape[0]
  emb = lax.dot_general(
      rows_ref[...], w_ref[...],
      (((1,), (1,)), ((), ())),
      preferred_element_type=jnp.float32,
  ) + b_ref[...]
  out_ref[:, :in_dim] = feats_ref[...]
  out_ref[:, in_dim:in_dim + d] = emb


def kernel(ids, feats, layer_idx, table, W, b):
  batch, in_dim = feats.shape
  d = table.shape[1]
  out_shape = jax.ShapeDtypeStruct((batch, in_dim + d), jnp.float32)

  gate = jnp.broadcast_to(
      jnp.asarray(layer_idx, jnp.int32).reshape(()), (_LANES,))
  rows = _make_gather(batch, table.shape[0], d)(
      ids.astype(jnp.int32), gate, table)

  grid = batch // _ROW_BLOCK
  out = pl.pallas_call(
      _fused_body,
      grid=(grid,),
      in_specs=[
          pl.BlockSpec((_ROW_BLOCK, in_dim), lambda i: (i, 0)),
          pl.BlockSpec((_ROW_BLOCK, d), lambda i: (i, 0)),
          pl.BlockSpec((d, d), lambda i: (0, 0)),
          pl.BlockSpec((1, d), lambda i: (0, 0)),
      ],
      out_specs=pl.BlockSpec((_ROW_BLOCK, in_dim + d), lambda i: (i, 0)),
      out_shape=out_shape,
  )(feats, rows, W, b.reshape(1, d))
  return out


# speedup vs baseline: 1.0205x; 1.0081x over previous
import functools

import jax
import jax.numpy as jnp
from jax import lax
from jax.experimental import pallas as pl
from jax.experimental.pallas import tpu as pltpu
from jax.experimental.pallas import tpu_sc as plsc

_NUM_CORES = 2
_NUM_SUBCORES = 16
_N_WORKERS = _NUM_CORES * _NUM_SUBCORES

_CHUNK = 800
_LANES = 16

_ROW_BLOCK = 10000


@functools.lru_cache(maxsize=None)
def _make_gather(batch, n_rows, d):
  n_chunks = batch // _CHUNK
  assert batch % _CHUNK == 0
  max_rounds = -(-n_chunks // _N_WORKERS)
  assert max_rounds >= 4 and n_chunks >= (max_rounds - 2) * _N_WORKERS
  mesh = plsc.VectorSubcoreMesh(core_axis_name="c", subcore_axis_name="s")

  @functools.partial(
      pl.kernel,
      mesh=mesh,
      compiler_params=pltpu.CompilerParams(use_tc_tiling_on_sc=False),
      out_type=jax.ShapeDtypeStruct((batch, d), jnp.float32),
      scratch_types=[
          pltpu.VMEM((_CHUNK,), jnp.int32),
          pltpu.VMEM((_CHUNK,), jnp.int32),
          pltpu.VMEM((_CHUNK, d), jnp.float32),
          pltpu.VMEM((_CHUNK, d), jnp.float32),
          pltpu.VMEM((_LANES,), jnp.int32),
          pltpu.SemaphoreType.DMA,
          pltpu.SemaphoreType.DMA,
          pltpu.SemaphoreType.DMA,
          pltpu.SemaphoreType.DMA,
      ],
  )
  def gather(ids_hbm, gate_hbm, table_hbm, out_hbm,
             idx_v0, idx_v1, rows_v0, rows_v1, gate_v,
             gsem0, gsem1, wsem0, wsem1):
    idx_v = (idx_v0, idx_v1)
    rows_v = (rows_v0, rows_v1)
    gsem = (gsem0, gsem1)
    wsem = (wsem0, wsem1)

    w = lax.axis_index("s") * _NUM_CORES + lax.axis_index("c")
    pltpu.sync_copy(gate_hbm, gate_v)
    use_ids = gate_v[...] > 0
    sentinel = jnp.full((_LANES,), n_rows - 1, jnp.int32)

    def chunk_off(j):
      return (w + _N_WORKERS * j) * _CHUNK

    def has_chunk(j):
      return w + _N_WORKERS * j < n_chunks

    def load_select(j, b):
      pltpu.sync_copy(ids_hbm.at[pl.ds(chunk_off(j), _CHUNK)], idx_v[b])
      for i in range(_CHUNK // _LANES):
        sl = pl.ds(i * _LANES, _LANES)
        idx_v[b][sl] = jnp.where(use_ids, idx_v[b][sl], sentinel)

    def g_start(b):
      pltpu.async_copy(table_hbm.at[idx_v[b]], rows_v[b], gsem[b])

    def g_wait(b):
      pltpu.make_async_copy(table_hbm.at[idx_v[b]], rows_v[b],
                            gsem[b]).wait()

    def w_dst(j):
      return out_hbm.at[pl.ds(chunk_off(j), _CHUNK)]

    def w_start(j, b):
      pltpu.async_copy(rows_v[b], w_dst(j), wsem[b])

    def w_wait(j, b):
      pltpu.make_async_copy(rows_v[b], w_dst(j), wsem[b]).wait()

    load_select(0, 0)
    g_start(0)
    load_select(1, 1)
    g_start(1)
    g_wait(0)
    w_start(0, 0)
    g_wait(1)
    w_start(1, 1)
    for j in range(2, max_rounds):
      b = j % 2
      cond = has_chunk(j)

      @pl.when(cond)
      def _prep():
        load_select(j, b)

      w_wait(j - 2, b)

      @pl.when(cond)
      def _fire():
        g_start(b)
        g_wait(b)
        w_start(j, b)

    for j in range(max_rounds - 2, max_rounds):
      b = j % 2

      @pl.when(has_chunk(j))
      def _drain():
        w_wait(j, b)

  return gather


def _fused_body(feats_ref, rows_ref, w_ref, b_ref, out_ref):
  in_dim = feats_ref.shape[1]
  d = w_ref.shape[0]
  emb = lax.dot_general(
      rows_ref[...], w_ref[...],
      (((1,), (1,)), ((), ())),
      preferred_element_type=jnp.float32,
  ) + b_ref[...]
  out_ref[:, :in_dim] = feats_ref[...]
  out_ref[:, in_dim:in_dim + d] = emb


def kernel(ids, feats, layer_idx, table, W, b):
  batch, in_dim = feats.shape
  d = table.shape[1]
  out_shape = jax.ShapeDtypeStruct((batch, in_dim + d), jnp.float32)

  gate = jnp.broadcast_to(
      jnp.asarray(layer_idx, jnp.int32).reshape(()), (_LANES,))
  rows = _make_gather(batch, table.shape[0], d)(
      ids.astype(jnp.int32), gate, table)

  grid = batch // _ROW_BLOCK
  out = pl.pallas_call(
      _fused_body,
      grid=(grid,),
      in_specs=[
          pl.BlockSpec((_ROW_BLOCK, in_dim), lambda i: (i, 0)),
          pl.BlockSpec((_ROW_BLOCK, d), lambda i: (i, 0)),
          pl.BlockSpec((d, d), lambda i: (0, 0)),
          pl.BlockSpec((1, d), lambda i: (0, 0)),
      ],
      out_specs=pl.BlockSpec((_ROW_BLOCK, in_dim + d), lambda i: (i, 0)),
      out_shape=out_shape,
  )(feats, rows, W, b.reshape(1, d))
  return out
